# baseline (speedup 1.0000x reference)
import jax
import jax.numpy as jnp
from jax.experimental import pallas as pl

B = 4096
D = 64
K = 8192
TB = 256
NT = B // TB
COMMITMENT_COST = 0.25
_DN = (((1,), (1,)), ((), ()))


def _rvq_kernel(x_ref, cb0_ref, cb1_ref, cb2_ref, qsum_ref, idx_ref, acc_ref):
    x = x_ref[...]
    r = x
    qsum = jnp.zeros_like(x)
    csum = jnp.zeros((), jnp.float32)
    for lvl, cb_ref in enumerate((cb0_ref, cb1_ref, cb2_ref)):
        cb = cb_ref[...]
        rs = jnp.sum(r * r, axis=1, keepdims=True)
        cs = jnp.sum(cb * cb, axis=1)
        rc = jax.lax.dot_general(r, cb, _DN, preferred_element_type=jnp.float32)
        d = (rs - 2.0 * rc) + cs[None, :]
        idx = jnp.argmin(d, axis=1).astype(jnp.int32)
        onehot = (jax.lax.broadcasted_iota(jnp.int32, (TB, K), 1)
                  == idx[:, None]).astype(jnp.float32)
        q = jax.lax.dot_general(onehot, cb, (((1,), (0,)), ((), ())),
                                precision=jax.lax.Precision.HIGHEST,
                                preferred_element_type=jnp.float32)
        idx_ref[lvl, :] = idx
        qsum = qsum + q
        csum = csum + jnp.sum((q - r) ** 2)
        r = r - q
    qsum_ref[...] = qsum
    recon = jnp.sum((qsum - x) ** 2)
    acc_ref[...] = jnp.stack([csum, recon]).reshape(1, 1, 2)


def kernel(x, cb0, cb1, cb2):
    qsum, idxs, acc = pl.pallas_call(
        _rvq_kernel,
        grid=(NT,),
        in_specs=[
            pl.BlockSpec((TB, D), lambda i: (i, 0)),
            pl.BlockSpec((K, D), lambda i: (0, 0)),
            pl.BlockSpec((K, D), lambda i: (0, 0)),
            pl.BlockSpec((K, D), lambda i: (0, 0)),
        ],
        out_specs=[
            pl.BlockSpec((TB, D), lambda i: (i, 0)),
            pl.BlockSpec((3, TB), lambda i: (0, i)),
            pl.BlockSpec((1, 1, 2), lambda i: (i, 0, 0)),
        ],
        out_shape=[
            jax.ShapeDtypeStruct((B, D), jnp.float32),
            jax.ShapeDtypeStruct((3, B), jnp.int32),
            jax.ShapeDtypeStruct((NT, 1, 2), jnp.float32),
        ],
    )(x, cb0, cb1, cb2)
    sums = jnp.sum(acc, axis=(0, 1))
    denom = jnp.float32(B * D)
    commit = sums[0] / denom
    recon = sums[1] / denom
    total = recon + COMMITMENT_COST * commit
    return (qsum, idxs, recon, commit, total)


# speedup vs baseline: 2.3573x; 2.3573x over previous
import jax
import jax.numpy as jnp
from jax.experimental import pallas as pl
from jax.experimental.pallas import tpu as pltpu

B = 4096
D = 64
K = 8192
TB = 256
NT = B // TB
COMMITMENT_COST = 0.25
_DN_T = (((1,), (1,)), ((), ()))
_DN = (((1,), (0,)), ((), ()))


def _split3(cb):
    hi = cb.astype(jnp.bfloat16)
    r1 = cb - hi.astype(jnp.float32)
    mid = r1.astype(jnp.bfloat16)
    r2 = r1 - mid.astype(jnp.float32)
    lo = r2.astype(jnp.bfloat16)
    return hi, mid, lo


def _rvq_kernel(x_ref, cb0_ref, cb1_ref, cb2_ref,
                h0_ref, m0_ref, l0_ref, h1_ref, m1_ref, l1_ref,
                h2_ref, m2_ref, l2_ref,
                qsum_ref, idx_ref, acc_ref):
    x = x_ref[...]
    r = x
    qsum = jnp.zeros_like(x)
    csum = jnp.zeros((), jnp.float32)
    levels = (
        (cb0_ref, h0_ref, m0_ref, l0_ref),
        (cb1_ref, h1_ref, m1_ref, l1_ref),
        (cb2_ref, h2_ref, m2_ref, l2_ref),
    )
    iota = jax.lax.broadcasted_iota(jnp.int32, (TB, K), 1)
    for lvl, (cb_ref, h_ref, m_ref, l_ref) in enumerate(levels):
        cb = cb_ref[...]
        rs = jnp.sum(r * r, axis=1, keepdims=True)
        cs = jnp.sum(cb * cb, axis=1)
        rc = jax.lax.dot_general(r, cb, _DN_T, preferred_element_type=jnp.float32)
        d = (rs - 2.0 * rc) + cs[None, :]
        dmin = jnp.min(d, axis=1, keepdims=True)
        idx = jnp.min(jnp.where(d == dmin, iota, K), axis=1).astype(jnp.int32)
        onehot = (iota == idx[:, None]).astype(jnp.float32).astype(jnp.bfloat16)
        q = ((jax.lax.dot_general(onehot, h_ref[...], _DN,
                                  preferred_element_type=jnp.float32)
              + jax.lax.dot_general(onehot, m_ref[...], _DN,
                                    preferred_element_type=jnp.float32))
             + jax.lax.dot_general(onehot, l_ref[...], _DN,
                                   preferred_element_type=jnp.float32))
        idx_ref[lvl, :] = idx
        qsum = qsum + q
        csum = csum + jnp.sum((q - r) ** 2)
        r = r - q
    qsum_ref[...] = qsum
    recon = jnp.sum((qsum - x) ** 2)
    acc_ref[...] = jnp.stack([csum, recon]).reshape(1, 1, 2)


def kernel(x, cb0, cb1, cb2):
    splits = _split3(cb0) + _split3(cb1) + _split3(cb2)
    cb_spec = pl.BlockSpec((K, D), lambda i: (0, 0))
    qsum, idxs, acc = pl.pallas_call(
        _rvq_kernel,
        grid=(NT,),
        in_specs=[pl.BlockSpec((TB, D), lambda i: (i, 0))] + [cb_spec] * 12,
        out_specs=[
            pl.BlockSpec((TB, D), lambda i: (i, 0)),
            pl.BlockSpec((3, TB), lambda i: (0, i)),
            pl.BlockSpec((1, 1, 2), lambda i: (i, 0, 0)),
        ],
        out_shape=[
            jax.ShapeDtypeStruct((B, D), jnp.float32),
            jax.ShapeDtypeStruct((3, B), jnp.int32),
            jax.ShapeDtypeStruct((NT, 1, 2), jnp.float32),
        ],
        compiler_params=pltpu.CompilerParams(
            dimension_semantics=("parallel",)),
    )(x, cb0, cb1, cb2, *splits)
    sums = jnp.sum(acc, axis=(0, 1))
    denom = jnp.float32(B * D)
    commit = sums[0] / denom
    recon = sums[1] / denom
    total = recon + COMMITMENT_COST * commit
    return (qsum, idxs, recon, commit, total)
